# baseline (speedup 1.0000x reference)
import functools

import jax
import jax.numpy as jnp
from jax import lax
from jax.experimental import pallas as pl
from jax.experimental.pallas import tpu as pltpu
from jax.experimental.pallas import tpu_sc as plsc

B = 1024
S = 200
V = 1000000
D = 64
L = 16
NC = 2
NS = 16
NW = NC * NS
BPW = B // NW
PACK = 2


def _newton_rsqrt(v):
    i = lax.bitcast_convert_type(v, jnp.int32)
    i = jnp.int32(0x5F3759DF) - lax.shift_right_arithmetic(i, 1)
    y = lax.bitcast_convert_type(i, jnp.float32)
    half = 0.5 * v
    for _ in range(4):
        y = y * (1.5 - half * y * y)
    return y


def _sc_gather_reduce(table, seq_flat, tag_flat):
    mesh = plsc.VectorSubcoreMesh(
        core_axis_name="c", subcore_axis_name="s",
        num_cores=NC, num_subcores=NS)

    @functools.partial(
        pl.kernel,
        out_type=(jax.ShapeDtypeStruct((B, D), jnp.float32),
                  jax.ShapeDtypeStruct((B, D), jnp.float32)),
        mesh=mesh,
        compiler_params=pltpu.CompilerParams(needs_layout_passes=False,
                                             use_tc_tiling_on_sc=True),
        scratch_types=[
            pltpu.VMEM((BPW * S,), jnp.int32),
            pltpu.VMEM((BPW * S,), jnp.int32),
            pltpu.VMEM((S, PACK * D), jnp.float32),
            pltpu.VMEM((S, PACK * D), jnp.float32),
            pltpu.VMEM((BPW, D), jnp.float32),
            pltpu.VMEM((BPW,), jnp.int32),
            pltpu.VMEM((BPW,), jnp.int32),
            pltpu.VMEM((BPW, PACK * D), jnp.float32),
            pltpu.VMEM((BPW, D), jnp.float32),
            pltpu.SemaphoreType.DMA,
            pltpu.SemaphoreType.DMA,
            pltpu.SemaphoreType.DMA,
        ],
    )
    def sc_kernel(table_hbm, seq_hbm, tag_hbm, acc_out, tag_out,
                  idx_v, idx2_v, rows0, rows1, acc_v,
                  tagidx_v, tagidx2_v, tagrows_v, tagout_v,
                  sem0, sem1, semt):
        tblp = table_hbm
        rows = (rows0, rows1)
        sems = (sem0, sem1)
        wid = lax.axis_index("s") * NC + lax.axis_index("c")
        base = wid * BPW

        pltpu.sync_copy(seq_hbm.at[pl.ds(base * S, BPW * S)], idx_v)
        pltpu.sync_copy(tag_hbm.at[pl.ds(base, BPW)], tagidx_v)

        def to_block(v):
            return jnp.where(v >= VH, v - VH, v)

        def shift_step(i, _):
            off = pl.multiple_of(i * L, 8)
            idx2_v[pl.ds(off, L)] = to_block(idx_v[pl.ds(off, L)])
            return 0
        lax.fori_loop(0, BPW * S // L, shift_step, 0)
        for i in range(BPW // L):
            tagidx2_v[pl.ds(i * L, L)] = to_block(tagidx_v[pl.ds(i * L, L)])

        pltpu.make_async_copy(tblp.at[tagidx2_v], tagrows_v, semt).start()

        def start_gather(j, b):
            off = pl.multiple_of(j * S, 8)
            pltpu.make_async_copy(
                tblp.at[idx2_v.at[pl.ds(off, S)]], rows[b], sems[b]
            ).start()

        def wait_gather(j, b):
            off = pl.multiple_of(j * S, 8)
            pltpu.make_async_copy(
                tblp.at[idx2_v.at[pl.ds(off, S)]], rows[b], sems[b]
            ).wait()

        def process_batch(j, b):
            wait_gather(j, b)
            rref = rows[b]
            ibase = pl.multiple_of(j * S, 8)

            def one_row(rr, iv, lane, carry, rref=rref):
                a0, a1, a2, a3 = carry
                idx = iv[lane]
                hi = idx >= VH
                x0 = jnp.where(hi, rref[rr, pl.ds(D, L)],
                               rref[rr, pl.ds(0, L)])
                x1 = jnp.where(hi, rref[rr, pl.ds(D + L, L)],
                               rref[rr, pl.ds(L, L)])
                x2 = jnp.where(hi, rref[rr, pl.ds(D + 2 * L, L)],
                               rref[rr, pl.ds(2 * L, L)])
                x3 = jnp.where(hi, rref[rr, pl.ds(D + 3 * L, L)],
                               rref[rr, pl.ds(3 * L, L)])
                s1 = jnp.sum(x0 + x1 + x2 + x3)
                s2 = jnp.sum(x0 * x0 + x1 * x1 + x2 * x2 + x3 * x3)
                u = s1 * (1.0 / D)
                var = jnp.maximum(s2 * (1.0 / D) - u * u, 0.0) + 1e-12
                rs = _newton_rsqrt(var)
                w = jnp.where(idx > 0, rs, 0.0)
                uw = u * w
                return (a0 + (x0 * w - uw), a1 + (x1 * w - uw),
                        a2 + (x2 * w - uw), a3 + (x3 * w - uw))

            def group_step(g, carry, rref=rref, ibase=ibase):
                base_r = g * L
                iv = idx_v[pl.ds(pl.multiple_of(ibase + base_r, 8), L)]
                for k in range(L):
                    carry = one_row(base_r + k, iv, k, carry, rref=rref)
                return carry

            zero = jnp.zeros((L,), jnp.float32)
            carry = lax.fori_loop(
                0, S // L, group_step, (zero, zero, zero, zero))
            iv_t = idx_v[pl.ds(pl.multiple_of(ibase + S - L, 8), L)]
            for k in range(S - (S // L) * L):
                carry = one_row((S // L) * L + k, iv_t, L - (S % L) + k, carry,
                                rref=rref)
            a0, a1, a2, a3 = carry
            acc_v[j, pl.ds(0, L)] = a0
            acc_v[j, pl.ds(L, L)] = a1
            acc_v[j, pl.ds(2 * L, L)] = a2
            acc_v[j, pl.ds(3 * L, L)] = a3

        start_gather(0, 0)
        start_gather(1, 1)

        def ring_step(i, _):
            j = i * 2
            process_batch(j, 0)
            start_gather(j + 2, 0)
            process_batch(j + 1, 1)
            start_gather(j + 3, 1)
            return 0

        lax.fori_loop(0, (BPW - 2) // 2, ring_step, 0)
        process_batch(BPW - 2, 0)
        process_batch(BPW - 1, 1)

        pltpu.make_async_copy(tblp.at[tagidx2_v], tagrows_v, semt).wait()
        tivs = [tagidx_v[pl.ds(i * L, L)] for i in range(BPW // L)]
        for j in range(BPW):
            tj = tivs[j // L][j % L]
            thi = tj >= VH
            for c in range(D // L):
                tagout_v[j, pl.ds(c * L, L)] = jnp.where(
                    thi, tagrows_v[j, pl.ds(D + c * L, L)],
                    tagrows_v[j, pl.ds(c * L, L)])

        pltpu.sync_copy(acc_v, acc_out.at[pl.ds(base, BPW)])
        pltpu.sync_copy(tagout_v, tag_out.at[pl.ds(base, BPW)])

    return sc_kernel(table, seq_flat, tag_flat)


PB = 2048
NBLK = 245
VH = NBLK * PB


def _tc_pack_transpose(emb_t):
    grid = NBLK
    dn = (((0,), (0,)), ((), ()))

    def body(lo_ref, hi_ref, eye_ref, out_ref):
        eye = eye_ref[...]
        out_ref[:, 0:D] = lax.dot_general(
            lo_ref[...], eye, dn, preferred_element_type=jnp.float32)
        out_ref[:, D:PACK * D] = lax.dot_general(
            hi_ref[...], eye, dn, preferred_element_type=jnp.float32)

    return pl.pallas_call(
        body,
        grid=(grid,),
        in_specs=[pl.BlockSpec((D, PB), lambda i: (0, i)),
                  pl.BlockSpec((D, PB),
                               lambda i: (0, jnp.minimum(i + NBLK, V // PB))),
                  pl.BlockSpec((D, D), lambda i: (0, 0))],
        out_specs=pl.BlockSpec((PB, PACK * D), lambda i: (i, 0)),
        out_shape=jax.ShapeDtypeStruct((VH, PACK * D), jnp.float32),
    )(emb_t, emb_t, jnp.eye(D, dtype=jnp.float32))


def _tc_finish(accz, sequence, lw, lb, W, tag_emb):
    def body(acc_ref, seq_ref, lw_ref, lb_ref, w_ref, tag_ref, out_ref):
        cnt = jnp.sum((seq_ref[...] > 0).astype(jnp.float32),
                      axis=1, keepdims=True)
        y = lw_ref[...] * acc_ref[...] + lb_ref[...] * cnt
        y = jnp.dot(y, w_ref[...], preferred_element_type=jnp.float32)
        out_ref[...] = y / (cnt + 1e-6) + tag_ref[...]

    return pl.pallas_call(
        body,
        out_shape=jax.ShapeDtypeStruct((B, D), jnp.float32),
    )(accz, sequence, lw, lb, W, tag_emb)


def kernel(emb_table, W, ln_weight, ln_bias, shared_s1, shared_s2,
           sequence, tag):
    seq_flat = sequence.reshape(-1)
    tag_flat = tag.reshape(-1)
    packed = _tc_pack_transpose(emb_table.T)
    accz, tag_emb = _sc_gather_reduce(packed, seq_flat, tag_flat)
    rep = _tc_finish(accz, sequence, ln_weight.reshape(1, D),
                     ln_bias.reshape(1, D), W, tag_emb)
    return rep, jnp.zeros((), jnp.float32)


# speedup vs baseline: 1.1541x; 1.1541x over previous
import functools

import jax
import jax.numpy as jnp
from jax import lax
from jax.experimental import pallas as pl
from jax.experimental.pallas import tpu as pltpu
from jax.experimental.pallas import tpu_sc as plsc

B = 1024
S = 200
V = 1000000
D = 64
L = 16
NC = 2
NS = 16
NW = NC * NS
BPW = B // NW
PACK = 2


def _newton_rsqrt(v):
    i = lax.bitcast_convert_type(v, jnp.int32)
    i = jnp.int32(0x5F3759DF) - lax.shift_right_arithmetic(i, 1)
    y = lax.bitcast_convert_type(i, jnp.float32)
    half = 0.5 * v
    for _ in range(3):
        y = y * (1.5 - half * y * y)
    return y


def _sc_gather_reduce(table, seq_flat, tag_flat):
    mesh = plsc.VectorSubcoreMesh(
        core_axis_name="c", subcore_axis_name="s",
        num_cores=NC, num_subcores=NS)

    @functools.partial(
        pl.kernel,
        out_type=(jax.ShapeDtypeStruct((B, D), jnp.float32),
                  jax.ShapeDtypeStruct((B, D), jnp.float32)),
        mesh=mesh,
        compiler_params=pltpu.CompilerParams(needs_layout_passes=False,
                                             use_tc_tiling_on_sc=True),
        scratch_types=[
            pltpu.VMEM((BPW * S,), jnp.int32),
            pltpu.VMEM((BPW * S,), jnp.int32),
            pltpu.VMEM((S, PACK * D), jnp.float32),
            pltpu.VMEM((S, PACK * D), jnp.float32),
            pltpu.VMEM((BPW, D), jnp.float32),
            pltpu.VMEM((BPW,), jnp.int32),
            pltpu.VMEM((BPW,), jnp.int32),
            pltpu.VMEM((BPW, PACK * D), jnp.float32),
            pltpu.VMEM((BPW, D), jnp.float32),
            pltpu.SemaphoreType.DMA,
            pltpu.SemaphoreType.DMA,
            pltpu.SemaphoreType.DMA,
        ],
    )
    def sc_kernel(table_hbm, seq_hbm, tag_hbm, acc_out, tag_out,
                  idx_v, idx2_v, rows0, rows1, acc_v,
                  tagidx_v, tagidx2_v, tagrows_v, tagout_v,
                  sem0, sem1, semt):
        tblp = table_hbm
        rows = (rows0, rows1)
        sems = (sem0, sem1)
        wid = lax.axis_index("s") * NC + lax.axis_index("c")
        base = wid * BPW

        pltpu.sync_copy(seq_hbm.at[pl.ds(base * S, BPW * S)], idx_v)
        pltpu.sync_copy(tag_hbm.at[pl.ds(base, BPW)], tagidx_v)

        def to_block(v):
            return jnp.where(v >= VH, v - VH, v)

        def shift_step(i, _):
            off = pl.multiple_of(i * L, 8)
            idx2_v[pl.ds(off, L)] = to_block(idx_v[pl.ds(off, L)])
            return 0
        lax.fori_loop(0, BPW * S // L, shift_step, 0)
        for i in range(BPW // L):
            tagidx2_v[pl.ds(i * L, L)] = to_block(tagidx_v[pl.ds(i * L, L)])

        pltpu.make_async_copy(tblp.at[tagidx2_v], tagrows_v, semt).start()

        def start_gather(j, b):
            off = pl.multiple_of(j * S, 8)
            pltpu.make_async_copy(
                tblp.at[idx2_v.at[pl.ds(off, S)]], rows[b], sems[b]
            ).start()

        def wait_gather(j, b):
            off = pl.multiple_of(j * S, 8)
            pltpu.make_async_copy(
                tblp.at[idx2_v.at[pl.ds(off, S)]], rows[b], sems[b]
            ).wait()

        def process_batch(j, b):
            wait_gather(j, b)
            rref = rows[b]
            ibase = pl.multiple_of(j * S, 8)

            def one_row(rr, iv, lane, carry, rref=rref):
                a0, a1, a2, a3 = carry
                idx = iv[lane]
                hoff = pl.multiple_of(
                    jnp.where(idx >= VH, D, 0), D)
                x0 = rref[rr, pl.ds(hoff, L)]
                x1 = rref[rr, pl.ds(hoff + L, L)]
                x2 = rref[rr, pl.ds(hoff + 2 * L, L)]
                x3 = rref[rr, pl.ds(hoff + 3 * L, L)]
                s1 = jnp.sum(x0 + x1 + x2 + x3)
                s2 = jnp.sum(x0 * x0 + x1 * x1 + x2 * x2 + x3 * x3)
                u = s1 * (1.0 / D)
                var = jnp.maximum(s2 * (1.0 / D) - u * u, 0.0) + 1e-12
                rs = _newton_rsqrt(var)
                w = jnp.where(idx > 0, rs, 0.0)
                uw = u * w
                return (a0 + (x0 * w - uw), a1 + (x1 * w - uw),
                        a2 + (x2 * w - uw), a3 + (x3 * w - uw))

            def group_step(g, carry, rref=rref, ibase=ibase):
                base_r = g * L
                iv = idx_v[pl.ds(pl.multiple_of(ibase + base_r, 8), L)]
                for k in range(L):
                    carry = one_row(base_r + k, iv, k, carry, rref=rref)
                return carry

            zero = jnp.zeros((L,), jnp.float32)
            carry = lax.fori_loop(
                0, S // L, group_step, (zero, zero, zero, zero))
            iv_t = idx_v[pl.ds(pl.multiple_of(ibase + S - L, 8), L)]
            for k in range(S - (S // L) * L):
                carry = one_row((S // L) * L + k, iv_t, L - (S % L) + k, carry,
                                rref=rref)
            a0, a1, a2, a3 = carry
            acc_v[j, pl.ds(0, L)] = a0
            acc_v[j, pl.ds(L, L)] = a1
            acc_v[j, pl.ds(2 * L, L)] = a2
            acc_v[j, pl.ds(3 * L, L)] = a3

        start_gather(0, 0)
        start_gather(1, 1)

        def ring_step(i, _):
            j = i * 2
            process_batch(j, 0)
            start_gather(j + 2, 0)
            process_batch(j + 1, 1)
            start_gather(j + 3, 1)
            return 0

        lax.fori_loop(0, (BPW - 2) // 2, ring_step, 0)
        process_batch(BPW - 2, 0)
        process_batch(BPW - 1, 1)

        pltpu.make_async_copy(tblp.at[tagidx2_v], tagrows_v, semt).wait()
        tivs = [tagidx_v[pl.ds(i * L, L)] for i in range(BPW // L)]
        for j in range(BPW):
            tj = tivs[j // L][j % L]
            thi = tj >= VH
            for c in range(D // L):
                tagout_v[j, pl.ds(c * L, L)] = jnp.where(
                    thi, tagrows_v[j, pl.ds(D + c * L, L)],
                    tagrows_v[j, pl.ds(c * L, L)])

        pltpu.sync_copy(acc_v, acc_out.at[pl.ds(base, BPW)])
        pltpu.sync_copy(tagout_v, tag_out.at[pl.ds(base, BPW)])

    return sc_kernel(table, seq_flat, tag_flat)


PB = 4096
NBLK = 123
VH = NBLK * PB


def _tc_pack_transpose(emb_t):
    grid = NBLK
    dn = (((0,), (0,)), ((), ()))

    def body(lo_ref, hi_ref, eye_ref, out_ref):
        eye = eye_ref[...]
        out_ref[:, 0:D] = lax.dot_general(
            lo_ref[...], eye, dn, preferred_element_type=jnp.float32)
        out_ref[:, D:PACK * D] = lax.dot_general(
            hi_ref[...], eye, dn, preferred_element_type=jnp.float32)

    return pl.pallas_call(
        body,
        grid=(grid,),
        in_specs=[pl.BlockSpec((D, PB), lambda i: (0, i)),
                  pl.BlockSpec((D, PB),
                               lambda i: (0, jnp.minimum(i + NBLK, V // PB))),
                  pl.BlockSpec((D, D), lambda i: (0, 0))],
        out_specs=pl.BlockSpec((PB, PACK * D), lambda i: (i, 0)),
        out_shape=jax.ShapeDtypeStruct((VH, PACK * D), jnp.float32),
    )(emb_t, emb_t, jnp.eye(D, dtype=jnp.float32))


def _tc_finish(accz, sequence, lw, lb, W, tag_emb):
    def body(acc_ref, seq_ref, lw_ref, lb_ref, w_ref, tag_ref, out_ref):
        cnt = jnp.sum((seq_ref[...] > 0).astype(jnp.float32),
                      axis=1, keepdims=True)
        y = lw_ref[...] * acc_ref[...] + lb_ref[...] * cnt
        y = jnp.dot(y, w_ref[...], preferred_element_type=jnp.float32)
        out_ref[...] = y / (cnt + 1e-6) + tag_ref[...]

    return pl.pallas_call(
        body,
        out_shape=jax.ShapeDtypeStruct((B, D), jnp.float32),
    )(accz, sequence, lw, lb, W, tag_emb)


def kernel(emb_table, W, ln_weight, ln_bias, shared_s1, shared_s2,
           sequence, tag):
    seq_flat = sequence.reshape(-1)
    tag_flat = tag.reshape(-1)
    packed = _tc_pack_transpose(emb_table.T)
    accz, tag_emb = _sc_gather_reduce(packed, seq_flat, tag_flat)
    rep = _tc_finish(accz, sequence, ln_weight.reshape(1, D),
                     ln_bias.reshape(1, D), W, tag_emb)
    return rep, jnp.zeros((), jnp.float32)
